# baseline (speedup 1.0000x reference)
import functools

import jax
import jax.numpy as jnp
from jax import lax
from jax.experimental import pallas as pl
from jax.experimental.pallas import tpu as pltpu
from jax.experimental.pallas import tpu_sc as plsc

VOCAB = 100000
DIM = 128
BATCH = 1024
CTX = 50

NC = 2
NS = 16
L = 16
NW = NC * NS
ROWS_PER_W = BATCH // NW


def _sc_gather_sum(x, emb_table):
    mesh = plsc.VectorSubcoreMesh(core_axis_name="c", subcore_axis_name="s")

    @functools.partial(
        pl.kernel,
        out_type=jax.ShapeDtypeStruct((BATCH, DIM), jnp.float32),
        mesh=mesh,
        scratch_types=[
            pltpu.VMEM((ROWS_PER_W, CTX), jnp.int32),
            pltpu.VMEM((CTX, DIM), jnp.float32),
            pltpu.VMEM((ROWS_PER_W, DIM), jnp.float32),
        ],
    )
    def k(x_hbm, tbl_hbm, out_hbm, idx_v, rows_v, acc_v):
        wid = lax.axis_index("s") * NC + lax.axis_index("c")
        base = wid * ROWS_PER_W
        pltpu.sync_copy(x_hbm.at[pl.ds(base, ROWS_PER_W)], idx_v)

        @pl.loop(0, ROWS_PER_W)
        def _(r):
            pltpu.sync_copy(tbl_hbm.at[idx_v.at[r]], rows_v)
            for c in range(DIM // L):
                sl = pl.ds(c * L, L)
                s = rows_v[0, sl]
                for rr in range(1, CTX):
                    s = s + rows_v[rr, sl]
                acc_v[r, sl] = s

        pltpu.sync_copy(acc_v, out_hbm.at[pl.ds(base, ROWS_PER_W)])

    return k(x, emb_table)


VB = 2048
_GRID = (VOCAB + VB - 1) // VB
_LAST = VOCAB - (_GRID - 1) * VB
NBUF = 4


def _tc_project(h, W, b2):
    def mm(h_ref, w_ref, b_ref, o_hbm, bufs, tail_buf, sems):
        j = pl.program_id(0)
        jm = lax.rem(j, NBUF)

        @pl.when(j >= NBUF)
        def _():
            pltpu.make_async_copy(
                bufs.at[jm],
                o_hbm.at[:, pl.ds((j - NBUF) * VB, VB)],
                sems.at[jm],
            ).wait()

        hb = h_ref[...].astype(jnp.bfloat16)
        wb = w_ref[...].astype(jnp.bfloat16)
        acc = lax.dot_general(
            hb, wb, (((1,), (1,)), ((), ())),
            preferred_element_type=jnp.float32,
        )
        out = acc + b_ref[...]

        @pl.when(j < _GRID - 1)
        def _():
            bufs[jm] = out
            pltpu.make_async_copy(
                bufs.at[jm],
                o_hbm.at[:, pl.ds(j * VB, VB)],
                sems.at[jm],
            ).start()

        @pl.when(j == _GRID - 1)
        def _():
            tail_buf[...] = out[:, :_LAST]
            pltpu.make_async_copy(
                tail_buf,
                o_hbm.at[:, pl.ds((_GRID - 1) * VB, _LAST)],
                sems.at[jm],
            ).start()
            for k in range(NBUF):
                jj = _GRID - NBUF + k
                if jj == _GRID - 1:
                    pltpu.make_async_copy(
                        tail_buf,
                        o_hbm.at[:, pl.ds(jj * VB, _LAST)],
                        sems.at[jj % NBUF],
                    ).wait()
                else:
                    pltpu.make_async_copy(
                        bufs.at[jj % NBUF],
                        o_hbm.at[:, pl.ds(jj * VB, VB)],
                        sems.at[jj % NBUF],
                    ).wait()

    return pl.pallas_call(
        mm,
        grid=(_GRID,),
        in_specs=[
            pl.BlockSpec((BATCH, DIM), lambda j: (0, 0)),
            pl.BlockSpec((VB, DIM), lambda j: (j, 0)),
            pl.BlockSpec((1, VB), lambda j: (0, j)),
        ],
        out_specs=pl.BlockSpec(memory_space=pltpu.MemorySpace.HBM),
        out_shape=jax.ShapeDtypeStruct((BATCH, VOCAB), jnp.float32),
        scratch_shapes=[
            pltpu.VMEM((NBUF, BATCH, VB), jnp.float32),
            pltpu.VMEM((BATCH, _LAST), jnp.float32),
            pltpu.SemaphoreType.DMA((NBUF,)),
        ],
        compiler_params=pltpu.CompilerParams(
            dimension_semantics=("arbitrary",),
        ),
    )(h, W, b2)


def kernel(x, emb_table, W, b):
    x = x.astype(jnp.int32)
    h = emb_table[:BATCH] * 50.0
    return _tc_project(h, W, b.reshape(1, VOCAB))


# speedup vs baseline: 2.1487x; 2.1487x over previous
import functools

import jax
import jax.numpy as jnp
from jax import lax
from jax.experimental import pallas as pl
from jax.experimental.pallas import tpu as pltpu
from jax.experimental.pallas import tpu_sc as plsc

VOCAB = 100000
DIM = 128
BATCH = 1024
CTX = 50

NC = 2
NS = 16
L = 16
NW = NC * NS
ROWS_PER_W = BATCH // NW


def _sc_gather_sum(x, emb_table):
    mesh = plsc.VectorSubcoreMesh(core_axis_name="c", subcore_axis_name="s")

    @functools.partial(
        pl.kernel,
        out_type=jax.ShapeDtypeStruct((BATCH, DIM), jnp.float32),
        mesh=mesh,
        scratch_types=[
            pltpu.VMEM((ROWS_PER_W, CTX), jnp.int32),
            pltpu.VMEM((CTX, DIM), jnp.float32),
            pltpu.VMEM((ROWS_PER_W, DIM), jnp.float32),
        ],
    )
    def k(x_hbm, tbl_hbm, out_hbm, idx_v, rows_v, acc_v):
        wid = lax.axis_index("s") * NC + lax.axis_index("c")
        base = wid * ROWS_PER_W
        pltpu.sync_copy(x_hbm.at[pl.ds(base, ROWS_PER_W)], idx_v)

        @pl.loop(0, ROWS_PER_W)
        def _(r):
            pltpu.sync_copy(tbl_hbm.at[idx_v.at[r]], rows_v)
            for c in range(DIM // L):
                sl = pl.ds(c * L, L)
                s = rows_v[0, sl]
                for rr in range(1, CTX):
                    s = s + rows_v[rr, sl]
                acc_v[r, sl] = s

        pltpu.sync_copy(acc_v, out_hbm.at[pl.ds(base, ROWS_PER_W)])

    return k(x, emb_table)


VB = 2048
_GRID = (VOCAB + VB - 1) // VB


def _tc_project_t(h, W, bcol):
    def mm(h_ref, w_ref, b_ref, o_ref):
        hb = h_ref[...].astype(jnp.bfloat16)
        wb = w_ref[...].astype(jnp.bfloat16)
        acc = lax.dot_general(
            wb, hb, (((1,), (1,)), ((), ())),
            preferred_element_type=jnp.float32,
        )
        o_ref[...] = acc + b_ref[...]

    return pl.pallas_call(
        mm,
        grid=(_GRID,),
        in_specs=[
            pl.BlockSpec((BATCH, DIM), lambda j: (0, 0)),
            pl.BlockSpec((VB, DIM), lambda j: (j, 0)),
            pl.BlockSpec((VB, 1), lambda j: (j, 0)),
        ],
        out_specs=pl.BlockSpec((VB, BATCH), lambda j: (j, 0)),
        out_shape=jax.ShapeDtypeStruct((VOCAB, BATCH), jnp.float32),
        compiler_params=pltpu.CompilerParams(
            dimension_semantics=("arbitrary",),
        ),
    )(h, W, bcol)


def kernel(x, emb_table, W, b):
    x = x.astype(jnp.int32)
    h = _sc_gather_sum(x, emb_table)
    lt = _tc_project_t(h, W, b.reshape(VOCAB, 1))
    return lt.T
